# Initial kernel scaffold; baseline (speedup 1.0000x reference)
import functools

import jax
import jax.numpy as jnp
from jax import lax
from jax.experimental import pallas as pl
from jax.experimental.pallas import tpu as pltpu
from jax.experimental.pallas import tpu_sc as plsc

_B, _L, _H, _V = 4096, 200, 128, 1000
_N = _B * _L
_NC, _NS = 2, 16
_NW = _NC * _NS
_TPW = _N // _NW
_C = 128
_NCHUNK = _TPW // _C

_mesh = plsc.VectorSubcoreMesh(core_axis_name="c", subcore_axis_name="s")


def _rsqrt_newton(v):
    bits = plsc.bitcast(v, jnp.int32)
    y = plsc.bitcast(jnp.int32(0x5F3759DF) - (bits >> 1), jnp.float32)
    half = v * 0.5
    for _ in range(3):
        y = y * (1.5 - half * y * y)
    return y


@functools.partial(
    pl.kernel,
    mesh=_mesh,
    out_type=jax.ShapeDtypeStruct((_N, _H), jnp.float32),
    scratch_types=[
        pltpu.VMEM((_C,), jnp.int32),
        pltpu.VMEM((_C, _H), jnp.float32),
        pltpu.VMEM((_C, _H), jnp.float32),
        pltpu.VMEM((_H,), jnp.float32),
        pltpu.VMEM((_H,), jnp.float32),
        pltpu.SemaphoreType.DMA,
    ],
)
def _sc_fused(emb_hbm, ids_hbm, table_hbm, gamma_hbm, beta_hbm, out_hbm,
              ids_v, emb_v, rows_v, gamma_v, beta_v, sem):
    wid = lax.axis_index("s") * _NC + lax.axis_index("c")
    base = wid * _TPW
    pltpu.sync_copy(gamma_hbm, gamma_v)
    pltpu.sync_copy(beta_hbm, beta_v)
    gs = tuple(gamma_v[pl.ds(16 * j, 16)] for j in range(8))
    bs = tuple(beta_v[pl.ds(16 * j, 16)] for j in range(8))

    def chunk_body(i, carry):
        off = base + i * _C
        pltpu.sync_copy(ids_hbm.at[pl.ds(off, _C)], ids_v)
        pltpu.sync_copy(emb_hbm.at[pl.ds(off, _C)], emb_v)
        pltpu.async_copy(table_hbm.at[ids_v], rows_v, sem).wait()

        def tok_body(t, tc):
            x = [emb_v[t, pl.ds(16 * j, 16)] + rows_v[t, pl.ds(16 * j, 16)]
                 for j in range(8)]
            s = x[0] + x[1]
            for j in range(2, 8):
                s = s + x[j]
            s2 = x[0] * x[0] + x[1] * x[1]
            for j in range(2, 8):
                s2 = s2 + x[j] * x[j]
            tot = jnp.sum(s)
            tot2 = jnp.sum(s2)
            mean = tot * (1.0 / _H)
            var = tot2 * (1.0 / _H) - mean * mean
            inv = _rsqrt_newton(jnp.broadcast_to(var + 1e-5, (16,)))
            meanv = jnp.broadcast_to(mean, (16,))
            for j in range(8):
                rows_v[t, pl.ds(16 * j, 16)] = (
                    (x[j] - meanv) * (inv * gs[j]) + bs[j])
            return tc

        lax.fori_loop(0, _C, tok_body, 0)
        pltpu.sync_copy(rows_v, out_hbm.at[pl.ds(off, _C)])
        return carry

    lax.fori_loop(0, _NCHUNK, chunk_body, 0)


def kernel(batch_mention_emb, mention_type_ids, table, gamma, beta):
    emb = batch_mention_emb.reshape(_N, _H)
    ids = mention_type_ids.reshape(_N).astype(jnp.int32)
    out = _sc_fused(emb, ids, table, gamma, beta)
    return out.reshape(_B, _L, _H)


# speedup vs baseline: 1.8336x; 1.8336x over previous
import functools

import jax
import jax.numpy as jnp
import numpy as np
from jax import lax
from jax.experimental import pallas as pl
from jax.experimental.pallas import tpu as pltpu
from jax.experimental.pallas import tpu_sc as plsc

_B, _L, _H, _V = 4096, 200, 128, 1000
_N = _B * _L
_NC, _NS = 2, 16
_NW = _NC * _NS
_TPW = _N // _NW
_C = 128
_NCHUNK = _TPW // _C

_mesh = plsc.VectorSubcoreMesh(core_axis_name="c", subcore_axis_name="s")

_GATHER_DNUMS = lax.GatherDimensionNumbers(
    offset_dims=(), collapsed_slice_dims=(0,), start_index_map=(0,))


def _perm(v, p):
    return lax.gather(v, p, _GATHER_DNUMS, slice_sizes=(1,),
                      mode=lax.GatherScatterMode.PROMISE_IN_BOUNDS)


def _xlane_sum(v):
    lanes = lax.iota(jnp.int32, 16)
    for k in (8, 4, 2, 1):
        p = (lanes ^ k).reshape(16, 1)
        v = v + _perm(v, p)
    return v


def _rsqrt_newton(v):
    bits = plsc.bitcast(v, jnp.int32)
    y = plsc.bitcast(jnp.int32(0x5F3759DF) - (bits >> 1), jnp.float32)
    half = v * 0.5
    for _ in range(3):
        y = y * (1.5 - half * y * y)
    return y


@functools.partial(
    pl.kernel,
    mesh=_mesh,
    out_type=jax.ShapeDtypeStruct((_N, _H), jnp.float32),
    compiler_params=pltpu.CompilerParams(needs_layout_passes=False),
    scratch_types=[
        pltpu.VMEM((_C,), jnp.int32),
        pltpu.VMEM((_C, _H), jnp.float32),
        pltpu.VMEM((_C, _H), jnp.float32),
        pltpu.VMEM((_H,), jnp.float32),
        pltpu.VMEM((_H,), jnp.float32),
        pltpu.SemaphoreType.DMA,
    ],
)
def _sc_fused(emb_hbm, ids_hbm, table_hbm, gamma_hbm, beta_hbm, out_hbm,
              ids_v, emb_v, rows_v, gamma_v, beta_v, sem):
    wid = lax.axis_index("s") * _NC + lax.axis_index("c")
    base = wid * _TPW
    pltpu.sync_copy(gamma_hbm, gamma_v)
    pltpu.sync_copy(beta_hbm, beta_v)
    gs = tuple(gamma_v[pl.ds(16 * j, 16)] for j in range(8))
    bs = tuple(beta_v[pl.ds(16 * j, 16)] for j in range(8))

    def chunk_body(i, carry):
        off = base + i * _C
        pltpu.sync_copy(ids_hbm.at[pl.ds(off, _C)], ids_v)
        pltpu.sync_copy(emb_hbm.at[pl.ds(off, _C)], emb_v)
        pltpu.async_copy(table_hbm.at[ids_v], rows_v, sem).wait()

        def tok_body(t, tc):
            x = [emb_v[t, pl.ds(16 * j, 16)] + rows_v[t, pl.ds(16 * j, 16)]
                 for j in range(8)]
            s = x[0] + x[1]
            for j in range(2, 8):
                s = s + x[j]
            s2 = x[0] * x[0] + x[1] * x[1]
            for j in range(2, 8):
                s2 = s2 + x[j] * x[j]
            meanv = _xlane_sum(s) * (1.0 / _H)
            var = _xlane_sum(s2) * (1.0 / _H) - meanv * meanv
            inv = _rsqrt_newton(var + 1e-5)
            for j in range(8):
                rows_v[t, pl.ds(16 * j, 16)] = (
                    (x[j] - meanv) * (inv * gs[j]) + bs[j])
            return tc

        lax.fori_loop(0, _C, tok_body, 0)
        pltpu.sync_copy(rows_v, out_hbm.at[pl.ds(off, _C)])
        return carry

    lax.fori_loop(0, _NCHUNK, chunk_body, 0)


def kernel(batch_mention_emb, mention_type_ids, table, gamma, beta):
    emb = batch_mention_emb.reshape(_N, _H)
    ids = mention_type_ids.reshape(_N).astype(jnp.int32)
    out = _sc_fused(emb, ids, table, gamma, beta)
    return out.reshape(_B, _L, _H)


# speedup vs baseline: 6.6530x; 3.6284x over previous
import functools

import jax
import jax.numpy as jnp
from jax import lax
from jax.experimental import pallas as pl
from jax.experimental.pallas import tpu as pltpu
from jax.experimental.pallas import tpu_sc as plsc

_B, _L, _H, _V = 4096, 200, 128, 1000
_N = _B * _L
_NC, _NS = 2, 16
_NW = _NC * _NS
_TPW = _N // _NW
_C = 128
_NCHUNK = _TPW // _C

_mesh = plsc.VectorSubcoreMesh(core_axis_name="c", subcore_axis_name="s")

_GATHER_DNUMS = lax.GatherDimensionNumbers(
    offset_dims=(), collapsed_slice_dims=(0,), start_index_map=(0,))


def _perm(v, p):
    return lax.gather(v, p, _GATHER_DNUMS, slice_sizes=(1,),
                      mode=lax.GatherScatterMode.PROMISE_IN_BOUNDS)


def _xlane_sum(v):
    lanes = lax.iota(jnp.int32, 16)
    for k in (8, 4, 2, 1):
        p = (lanes ^ k).reshape(16, 1)
        v = v + _perm(v, p)
    return v


def _rsqrt_newton(v):
    bits = plsc.bitcast(v, jnp.int32)
    y = plsc.bitcast(jnp.int32(0x5F3759DF) - (bits >> 1), jnp.float32)
    half = v * 0.5
    for _ in range(2):
        y = y * (1.5 - half * y * y)
    return y


@functools.partial(
    pl.kernel,
    mesh=_mesh,
    out_type=jax.ShapeDtypeStruct((_N, _H), jnp.float32),
    compiler_params=pltpu.CompilerParams(needs_layout_passes=False),
    scratch_types=[
        pltpu.VMEM((_TPW,), jnp.int32),
        pltpu.VMEM((2, _C, _H), jnp.float32),
        pltpu.VMEM((2, _C, _H), jnp.float32),
        pltpu.VMEM((2, _C, _H), jnp.float32),
        pltpu.VMEM((_H,), jnp.float32),
        pltpu.VMEM((_H,), jnp.float32),
        pltpu.SemaphoreType.DMA,
        pltpu.SemaphoreType.DMA,
        pltpu.SemaphoreType.DMA,
        pltpu.SemaphoreType.DMA,
        pltpu.SemaphoreType.DMA,
        pltpu.SemaphoreType.DMA,
    ],
)
def _sc_fused(emb_hbm, ids_hbm, table_hbm, gamma_hbm, beta_hbm, out_hbm,
              ids_v, emb_v, rows_v, out_v, gamma_v, beta_v,
              sem_e0, sem_e1, sem_g0, sem_g1, sem_o0, sem_o1):
    wid = lax.axis_index("s") * _NC + lax.axis_index("c")
    base = wid * _TPW
    sem_e = (sem_e0, sem_e1)
    sem_g = (sem_g0, sem_g1)
    sem_o = (sem_o0, sem_o1)

    pltpu.sync_copy(gamma_hbm, gamma_v)
    pltpu.sync_copy(beta_hbm, beta_v)
    pltpu.sync_copy(ids_hbm.at[pl.ds(base, _TPW)], ids_v)
    gs = tuple(gamma_v[pl.ds(16 * j, 16)] for j in range(8))
    bs = tuple(beta_v[pl.ds(16 * j, 16)] for j in range(8))

    def _emb_copy(i, b):
        return pltpu.make_async_copy(
            emb_hbm.at[pl.ds(base + i * _C, _C)], emb_v.at[b], sem_e[b])

    def _gather_copy(i, b):
        return pltpu.make_async_copy(
            table_hbm.at[ids_v.at[pl.ds(i * _C, _C)]], rows_v.at[b],
            sem_g[b])

    def _out_copy(i, b):
        return pltpu.make_async_copy(
            out_v.at[b], out_hbm.at[pl.ds(base + i * _C, _C)], sem_o[b])

    def _compute_chunk(b):
        eb, rb, ob = emb_v.at[b], rows_v.at[b], out_v.at[b]

        @plsc.parallel_loop(0, _C, unroll=2)
        def _tok(t):
            x = [eb[t, pl.ds(16 * j, 16)] + rb[t, pl.ds(16 * j, 16)]
                 for j in range(8)]
            s = (x[0] + x[1]) + (x[2] + x[3]) + ((x[4] + x[5]) + (x[6] + x[7]))
            s2 = ((x[0] * x[0] + x[1] * x[1]) + (x[2] * x[2] + x[3] * x[3])
                  + ((x[4] * x[4] + x[5] * x[5])
                     + (x[6] * x[6] + x[7] * x[7])))
            meanv = _xlane_sum(s) * (1.0 / _H)
            var = _xlane_sum(s2) * (1.0 / _H) - meanv * meanv
            inv = _rsqrt_newton(var + 1e-5)
            for j in range(8):
                ob[t, pl.ds(16 * j, 16)] = (x[j] - meanv) * (inv * gs[j]) + bs[j]

    _emb_copy(0, 0).start()
    _gather_copy(0, 0).start()

    def outer(g, carry):
        for b in (0, 1):
            i = 2 * g + b

            @pl.when(g >= 1)
            def _wait_out():
                _out_copy(i - 2, b).wait()

            if b == 0:
                _emb_copy(i + 1, 1).start()
                _gather_copy(i + 1, 1).start()
            else:
                @pl.when(g < _NCHUNK // 2 - 1)
                def _issue_next():
                    _emb_copy(i + 1, 0).start()
                    _gather_copy(i + 1, 0).start()

            _emb_copy(i, b).wait()
            _gather_copy(i, b).wait()
            _compute_chunk(b)
            _out_copy(i, b).start()
        return carry

    lax.fori_loop(0, _NCHUNK // 2, outer, 0)
    _out_copy(_NCHUNK - 2, 0).wait()
    _out_copy(_NCHUNK - 1, 1).wait()


def kernel(batch_mention_emb, mention_type_ids, table, gamma, beta):
    emb = batch_mention_emb.reshape(_N, _H)
    ids = mention_type_ids.reshape(_N).astype(jnp.int32)
    out = _sc_fused(emb, ids, table, gamma, beta)
    return out.reshape(_B, _L, _H)


# speedup vs baseline: 6.7517x; 1.0148x over previous
import functools

import jax
import jax.numpy as jnp
from jax import lax
from jax.experimental import pallas as pl
from jax.experimental.pallas import tpu as pltpu
from jax.experimental.pallas import tpu_sc as plsc

_B, _L, _H, _V = 4096, 200, 128, 1000
_N = _B * _L
_NC, _NS = 2, 16
_NW = _NC * _NS
_TPW = _N // _NW
_C = 128
_NCHUNK = _TPW // _C
_NX = 4

_mesh = plsc.VectorSubcoreMesh(core_axis_name="c", subcore_axis_name="s")


def _xlane_sum(v):
    return jnp.broadcast_to(jnp.sum(v), (16,))


def _rsqrt_newton(v):
    bits = plsc.bitcast(v, jnp.int32)
    y = plsc.bitcast(jnp.int32(0x5F3759DF) - (bits >> 1), jnp.float32)
    half = v * 0.5
    for _ in range(2):
        y = y * (1.5 - half * y * y)
    return y


@functools.partial(
    pl.kernel,
    mesh=_mesh,
    out_type=jax.ShapeDtypeStruct((_N, _H), jnp.float32),
    compiler_params=pltpu.CompilerParams(needs_layout_passes=False),
    scratch_types=[
        pltpu.VMEM((_TPW,), jnp.int32),
        pltpu.VMEM((_NX, _C, _H), jnp.float32),
        pltpu.VMEM((2, _C, _H), jnp.float32),
        pltpu.VMEM((_H,), jnp.float32),
        pltpu.VMEM((_H,), jnp.float32),
        pltpu.SemaphoreType.DMA,
        pltpu.SemaphoreType.DMA,
        pltpu.SemaphoreType.DMA,
        pltpu.SemaphoreType.DMA,
        pltpu.SemaphoreType.DMA,
        pltpu.SemaphoreType.DMA,
        pltpu.SemaphoreType.DMA,
        pltpu.SemaphoreType.DMA,
        pltpu.SemaphoreType.DMA,
        pltpu.SemaphoreType.DMA,
    ],
)
def _sc_fused(emb_hbm, ids_hbm, table_hbm, gamma_hbm, beta_hbm, out_hbm,
              ids_v, x_v, out_v, gamma_v, beta_v,
              sem_e0, sem_e1, sem_e2, sem_e3,
              sem_a0, sem_a1, sem_a2, sem_a3, sem_o0, sem_o1):
    wid = lax.axis_index("s") * _NC + lax.axis_index("c")
    base = wid * _TPW
    sem_e = (sem_e0, sem_e1, sem_e2, sem_e3)
    sem_a = (sem_a0, sem_a1, sem_a2, sem_a3)
    sem_o = (sem_o0, sem_o1)

    pltpu.sync_copy(gamma_hbm, gamma_v)
    pltpu.sync_copy(beta_hbm, beta_v)
    pltpu.sync_copy(ids_hbm.at[pl.ds(base, _TPW)], ids_v)
    gs = tuple(gamma_v[pl.ds(16 * j, 16)] for j in range(8))
    bs = tuple(beta_v[pl.ds(16 * j, 16)] for j in range(8))

    def _emb_copy(i, b):
        return pltpu.make_async_copy(
            emb_hbm.at[pl.ds(base + i * _C, _C)], x_v.at[b], sem_e[b])

    def _gadd_copy(i, b):
        return pltpu.make_async_copy(
            table_hbm.at[ids_v.at[pl.ds(i * _C, _C)]], x_v.at[b], sem_a[b])

    def _out_copy(i, b):
        return pltpu.make_async_copy(
            out_v.at[b], out_hbm.at[pl.ds(base + i * _C, _C)], sem_o[b])

    def _compute_chunk(xu, ou):
        xb, ob = x_v.at[xu], out_v.at[ou]

        @plsc.parallel_loop(0, _C, unroll=1)
        def _tok(t):
            x = [xb[t, pl.ds(16 * j, 16)] for j in range(8)]
            s = (x[0] + x[1]) + (x[2] + x[3]) + ((x[4] + x[5]) + (x[6] + x[7]))
            s2 = ((x[0] * x[0] + x[1] * x[1]) + (x[2] * x[2] + x[3] * x[3])
                  + ((x[4] * x[4] + x[5] * x[5])
                     + (x[6] * x[6] + x[7] * x[7])))
            meanv = _xlane_sum(s) * (1.0 / _H)
            var = _xlane_sum(s2) * (1.0 / _H) - meanv * meanv
            inv = _rsqrt_newton(var + 1e-5)
            for j in range(8):
                ob[t, pl.ds(16 * j, 16)] = (x[j] - meanv) * (inv * gs[j]) + bs[j]

    _emb_copy(0, 0).start()
    _emb_copy(1, 1).start()
    _emb_copy(0, 0).wait()
    _gadd_copy(0, 0).start(add=True)

    def outer(g, carry):
        for u in range(4):
            i = 4 * g + u
            nu, nnu, ou = (u + 1) % 4, (u + 2) % 4, u % 2

            if u < 3:
                _emb_copy(i + 1, nu).wait()
                _gadd_copy(i + 1, nu).start(add=True)
            else:
                @pl.when(g < _NCHUNK // 4 - 1)
                def _issue_gadd():
                    _emb_copy(i + 1, nu).wait()
                    _gadd_copy(i + 1, nu).start(add=True)

            if u < 2:
                _emb_copy(i + 2, nnu).start()
            else:
                @pl.when(g < _NCHUNK // 4 - 1)
                def _issue_emb():
                    _emb_copy(i + 2, nnu).start()

            if u >= 2:
                _out_copy(i - 2, ou).wait()
            else:
                @pl.when(g >= 1)
                def _wait_out():
                    _out_copy(i - 2, ou).wait()

            _gadd_copy(i, u).wait()
            _compute_chunk(u, ou)
            _out_copy(i, ou).start()
        return carry

    lax.fori_loop(0, _NCHUNK // 4, outer, 0)
    _out_copy(_NCHUNK - 2, 0).wait()
    _out_copy(_NCHUNK - 1, 1).wait()


def kernel(batch_mention_emb, mention_type_ids, table, gamma, beta):
    emb = batch_mention_emb.reshape(_N, _H)
    ids = mention_type_ids.reshape(_N).astype(jnp.int32)
    out = _sc_fused(emb, ids, table, gamma, beta)
    return out.reshape(_B, _L, _H)
